# Initial kernel scaffold; baseline (speedup 1.0000x reference)
import jax
import jax.numpy as jnp
from jax.experimental import pallas as pl


def kernel(patch_centers_voxels, mri_affine, aal_affine, aal_data, region_embed):
    raise NotImplementedError("write your pallas kernel here")


# speedup vs baseline: 1.4500x; 1.4500x over previous
import functools

import jax
import jax.numpy as jnp
from jax import lax
from jax.experimental import pallas as pl
from jax.experimental.pallas import tpu as pltpu
from jax.experimental.pallas import tpu_sc as plsc

EMBED_DIM = 768
REGION_MAX = 116
NW = 32
L = 16
ROUND_C = 12582912.0


@functools.lru_cache(maxsize=None)
def _build_sc_kernel(D, H, W, n_pts):
    P = n_pts // NW
    CH = 64
    n_chunks = P // CH
    GID = 128
    mesh = plsc.VectorSubcoreMesh(core_axis_name="c", subcore_axis_name="s")

    @functools.partial(
        pl.kernel,
        mesh=mesh,
        out_type=jax.ShapeDtypeStruct((n_pts, EMBED_DIM), jnp.float32),
        scratch_types=[
            pltpu.VMEM((P,), jnp.float32),
            pltpu.VMEM((P,), jnp.float32),
            pltpu.VMEM((P,), jnp.float32),
            pltpu.VMEM((P,), jnp.int32),
            pltpu.VMEM((P,), jnp.int32),
            pltpu.VMEM((P,), jnp.float32),
            pltpu.VMEM((P,), jnp.int32),
            pltpu.VMEM((2, CH, EMBED_DIM), jnp.float32),
            pltpu.SemaphoreType.DMA,
            pltpu.SemaphoreType.DMA,
            pltpu.SemaphoreType.DMA,
            pltpu.SemaphoreType.DMA,
        ],
    )
    def sc_kernel(xs_hbm, ys_hbm, zs_hbm, aal_hbm, embed_hbm, out_hbm,
                  x_v, y_v, z_v, idx_v, val_v, reg_v, rid_v,
                  rows_v, sem_g0, sem_g1, sem_o, sem_a):
        wid = lax.axis_index("s") * 2 + lax.axis_index("c")
        base = wid * P

        pltpu.sync_copy(xs_hbm.at[pl.ds(base, P)], x_v)
        pltpu.sync_copy(ys_hbm.at[pl.ds(base, P)], y_v)
        pltpu.sync_copy(zs_hbm.at[pl.ds(base, P)], z_v)

        def coord_body(i, carry):
            sl = pl.ds(pl.multiple_of(i * L, L), L)
            fx = (x_v[sl] + ROUND_C) - ROUND_C
            fy = (y_v[sl] + ROUND_C) - ROUND_C
            fz = (z_v[sl] + ROUND_C) - ROUND_C
            xi = fx.astype(jnp.int32)
            yi = fy.astype(jnp.int32)
            zi = fz.astype(jnp.int32)
            valid = ((xi >= 0) & (xi < D) & (yi >= 0) & (yi < H)
                     & (zi >= 0) & (zi < W))
            xc = jnp.minimum(jnp.maximum(xi, 0), D - 1)
            yc = jnp.minimum(jnp.maximum(yi, 0), H - 1)
            zc = jnp.minimum(jnp.maximum(zi, 0), W - 1)
            idx_v[sl] = (xc * (H * W) + yc * W + zc)
            val_v[sl] = jnp.where(valid, 1, 0)
            return carry

        lax.fori_loop(0, P // L, coord_body, 0)

        atlas_copies = []
        for j in range(P // GID):
            sl = pl.ds(j * GID, GID)
            atlas_copies.append(
                pltpu.async_copy(aal_hbm.at[idx_v.at[sl]], reg_v.at[sl], sem_a))
        for cp in atlas_copies:
            cp.wait()

        def region_body(i, carry):
            sl = pl.ds(pl.multiple_of(i * L, L), L)
            r = reg_v[sl].astype(jnp.int32)
            ok = (r >= 0) & (r <= REGION_MAX) & (val_v[sl] > 0)
            rid_v[sl] = jnp.where(ok, r, 0)
            return carry

        lax.fori_loop(0, P // L, region_body, 0)

        gsems = [sem_g0, sem_g1]
        gathers = [None] * n_chunks
        writes = [None] * n_chunks

        def gather_chunk(t):
            sl = pl.ds(t * CH, CH)
            return pltpu.async_copy(embed_hbm.at[rid_v.at[sl]],
                                    rows_v.at[t % 2], gsems[t % 2])

        gathers[0] = gather_chunk(0)
        for t in range(n_chunks):
            if t + 1 < n_chunks:
                if t >= 1:
                    writes[t - 1].wait()
                gathers[t + 1] = gather_chunk(t + 1)
            gathers[t].wait()
            writes[t] = pltpu.async_copy(
                rows_v.at[t % 2], out_hbm.at[pl.ds(base + t * CH, CH)], sem_o)
        writes[n_chunks - 2].wait()
        writes[n_chunks - 1].wait()

    return sc_kernel


def kernel(patch_centers_voxels, mri_affine, aal_affine, aal_data, region_embed):
    Bb, Nn, _ = patch_centers_voxels.shape
    D, H, W = aal_data.shape
    n_pts = Bb * Nn
    ones = jnp.ones((Bb, Nn, 1), dtype=jnp.float32)
    voxel_homo = jnp.concatenate(
        [patch_centers_voxels.astype(jnp.float32), ones], axis=-1)
    world_coords = jnp.einsum('ij,bnj->bni', mri_affine, voxel_homo)
    inv_aal_affine = jnp.linalg.inv(aal_affine)
    aal_voxel_coords = jnp.einsum('ij,bnj->bni', inv_aal_affine, world_coords)
    coords = aal_voxel_coords[..., :3].reshape(n_pts, 3)
    aal_flat = aal_data.astype(jnp.float32).reshape(-1)
    out = _build_sc_kernel(D, H, W, n_pts)(
        coords[:, 0], coords[:, 1], coords[:, 2], aal_flat,
        region_embed.astype(jnp.float32))
    return out.reshape(Bb, Nn, EMBED_DIM)


# speedup vs baseline: 2.9655x; 2.0452x over previous
import functools

import jax
import jax.numpy as jnp
from jax import lax
from jax.experimental import pallas as pl
from jax.experimental.pallas import tpu as pltpu
from jax.experimental.pallas import tpu_sc as plsc

EMBED_DIM = 768
REGION_MAX = 116
NREG_PAD = 128
NW = 32
L = 16
ROUND_C = 12582912.0
BLK = 1024


@functools.lru_cache(maxsize=None)
def _build_rid_kernel(D, H, W, n_pts):
    P = n_pts // NW
    GID = 128
    mesh = plsc.VectorSubcoreMesh(core_axis_name="c", subcore_axis_name="s")

    @functools.partial(
        pl.kernel,
        mesh=mesh,
        out_type=jax.ShapeDtypeStruct((n_pts,), jnp.int32),
        scratch_types=[
            pltpu.VMEM((P,), jnp.float32),
            pltpu.VMEM((P,), jnp.float32),
            pltpu.VMEM((P,), jnp.float32),
            pltpu.VMEM((P,), jnp.int32),
            pltpu.VMEM((P,), jnp.int32),
            pltpu.VMEM((P,), jnp.float32),
            pltpu.VMEM((P,), jnp.int32),
            pltpu.SemaphoreType.DMA,
        ],
    )
    def rid_kernel(xs_hbm, ys_hbm, zs_hbm, aal_hbm, rid_hbm,
                   x_v, y_v, z_v, idx_v, val_v, reg_v, rid_v, sem_a):
        wid = lax.axis_index("s") * 2 + lax.axis_index("c")
        base = wid * P

        pltpu.sync_copy(xs_hbm.at[pl.ds(base, P)], x_v)
        pltpu.sync_copy(ys_hbm.at[pl.ds(base, P)], y_v)
        pltpu.sync_copy(zs_hbm.at[pl.ds(base, P)], z_v)

        def coord_body(i, carry):
            sl = pl.ds(pl.multiple_of(i * L, L), L)
            fx = (x_v[sl] + ROUND_C) - ROUND_C
            fy = (y_v[sl] + ROUND_C) - ROUND_C
            fz = (z_v[sl] + ROUND_C) - ROUND_C
            xi = fx.astype(jnp.int32)
            yi = fy.astype(jnp.int32)
            zi = fz.astype(jnp.int32)
            valid = ((xi >= 0) & (xi < D) & (yi >= 0) & (yi < H)
                     & (zi >= 0) & (zi < W))
            xc = jnp.minimum(jnp.maximum(xi, 0), D - 1)
            yc = jnp.minimum(jnp.maximum(yi, 0), H - 1)
            zc = jnp.minimum(jnp.maximum(zi, 0), W - 1)
            idx_v[sl] = (xc * (H * W) + yc * W + zc)
            val_v[sl] = jnp.where(valid, 1, 0)
            return carry

        lax.fori_loop(0, P // L, coord_body, 0)

        atlas_copies = []
        for j in range(P // GID):
            sl = pl.ds(j * GID, GID)
            atlas_copies.append(
                pltpu.async_copy(aal_hbm.at[idx_v.at[sl]], reg_v.at[sl], sem_a))
        for cp in atlas_copies:
            cp.wait()

        def region_body(i, carry):
            sl = pl.ds(pl.multiple_of(i * L, L), L)
            r = reg_v[sl].astype(jnp.int32)
            ok = (r >= 0) & (r <= REGION_MAX) & (val_v[sl] > 0)
            rid_v[sl] = jnp.where(ok, r, 0)
            return carry

        lax.fori_loop(0, P // L, region_body, 0)
        pltpu.sync_copy(rid_v, rid_hbm.at[pl.ds(base, P)])

    return rid_kernel


def _expand_body(rid_ref, tab_ref, out_ref):
    rid = rid_ref[0, 0, :]
    onehot = (rid[:, None]
              == lax.broadcasted_iota(jnp.int32, (BLK, NREG_PAD), 1))
    out_ref[...] = jnp.dot(onehot.astype(jnp.float32), tab_ref[...],
                           preferred_element_type=jnp.float32)


@functools.lru_cache(maxsize=None)
def _build_expand(n_pts):
    grid = n_pts // BLK
    return pl.pallas_call(
        _expand_body,
        grid=(grid,),
        in_specs=[
            pl.BlockSpec((1, 1, BLK), lambda i: (i, 0, 0)),
            pl.BlockSpec((NREG_PAD, EMBED_DIM), lambda i: (0, 0)),
        ],
        out_specs=pl.BlockSpec((BLK, EMBED_DIM), lambda i: (i, 0)),
        out_shape=jax.ShapeDtypeStruct((n_pts, EMBED_DIM), jnp.float32),
    )


def kernel(patch_centers_voxels, mri_affine, aal_affine, aal_data, region_embed):
    Bb, Nn, _ = patch_centers_voxels.shape
    D, H, W = aal_data.shape
    n_pts = Bb * Nn
    ones = jnp.ones((Bb, Nn, 1), dtype=jnp.float32)
    voxel_homo = jnp.concatenate(
        [patch_centers_voxels.astype(jnp.float32), ones], axis=-1)
    world_coords = jnp.einsum('ij,bnj->bni', mri_affine, voxel_homo)
    inv_aal_affine = jnp.linalg.inv(aal_affine)
    aal_voxel_coords = jnp.einsum('ij,bnj->bni', inv_aal_affine, world_coords)
    coords = aal_voxel_coords[..., :3].reshape(n_pts, 3)
    aal_flat = aal_data.astype(jnp.float32).reshape(-1)
    rid = _build_rid_kernel(D, H, W, n_pts)(
        coords[:, 0], coords[:, 1], coords[:, 2], aal_flat)
    tab = jnp.zeros((NREG_PAD, EMBED_DIM), jnp.float32).at[:REGION_MAX + 1].set(
        region_embed.astype(jnp.float32))
    out = _build_expand(n_pts)(rid.reshape(n_pts // BLK, 1, BLK), tab)
    return out.reshape(Bb, Nn, EMBED_DIM)
